# baseline (speedup 1.0000x reference)
import functools

import jax
import jax.numpy as jnp
from jax import lax
from jax.experimental import pallas as pl
from jax.experimental.pallas import tpu as pltpu
from jax.experimental.pallas import tpu_sc as plsc

BATCH = 16384
DIM = 64
LANES = 16
NUM_CORES = 2
NUM_SUBCORES = 16
NUM_WORKERS = NUM_CORES * NUM_SUBCORES
ROWS_PER_WORKER = BATCH // NUM_WORKERS
CHUNK = 128
NCHUNK = ROWS_PER_WORKER // CHUNK
NGROUP = ROWS_PER_WORKER // LANES

COPY_GRID = 8
COPY_BLOCK = BATCH // COPY_GRID


def _gather_body(table_hbm, idx_hbm, out_hbm, idx_v, rows_v, out_t_v, sem):
    wid = lax.axis_index("s") * NUM_CORES + lax.axis_index("c")
    base = wid * ROWS_PER_WORKER
    pltpu.sync_copy(idx_hbm.at[pl.ds(wid * NCHUNK, NCHUNK)], idx_v)
    gathers = [
        pltpu.async_copy(
            table_hbm.at[idx_v.at[j]],
            rows_v.at[pl.ds(j * CHUNK, CHUNK)],
            sem,
        )
        for j in range(NCHUNK)
    ]
    for g in gathers:
        g.wait()

    @plsc.parallel_loop(0, NGROUP)
    def _(g):
        rvec = g * LANES + lax.iota(jnp.int32, LANES)
        for d in range(DIM):
            dvec = jnp.full((LANES,), d, dtype=jnp.int32)
            vals = plsc.load_gather(rows_v, [rvec, dvec])
            out_t_v[d, pl.ds(g * LANES, LANES)] = vals

    pltpu.sync_copy(out_t_v, out_hbm.at[:, pl.ds(base, ROWS_PER_WORKER)])


_gather_t = functools.partial(
    pl.kernel,
    out_type=jax.ShapeDtypeStruct((DIM, BATCH), jnp.float32),
    mesh=plsc.VectorSubcoreMesh(core_axis_name="c", subcore_axis_name="s"),
    scratch_types=[
        pltpu.VMEM((NCHUNK, CHUNK), jnp.int32),
        pltpu.VMEM((ROWS_PER_WORKER, DIM), jnp.float32),
        pltpu.VMEM((DIM, ROWS_PER_WORKER), jnp.float32),
        pltpu.SemaphoreType.DMA,
    ],
    compiler_params=pltpu.CompilerParams(
        use_tc_tiling_on_sc=False, needs_layout_passes=False
    ),
)(_gather_body)


def _copy_body(sub_ref, obj_ref, sub_out_ref, obj_out_ref):
    sub_out_ref[...] = sub_ref[...]
    obj_out_ref[...] = obj_ref[...]


_passthrough_t = pl.pallas_call(
    _copy_body,
    grid=(COPY_GRID,),
    in_specs=[
        pl.BlockSpec((DIM, COPY_BLOCK), lambda i: (0, i)),
        pl.BlockSpec((DIM, COPY_BLOCK), lambda i: (0, i)),
    ],
    out_specs=[
        pl.BlockSpec((DIM, COPY_BLOCK), lambda i: (0, i)),
        pl.BlockSpec((DIM, COPY_BLOCK), lambda i: (0, i)),
    ],
    out_shape=(
        jax.ShapeDtypeStruct((DIM, BATCH), jnp.float32),
        jax.ShapeDtypeStruct((DIM, BATCH), jnp.float32),
    ),
)


def kernel(subject_embeddings, relation_ids, object_embeddings, relation_table):
    idx2d = relation_ids.astype(jnp.int32).reshape(NUM_WORKERS * NCHUNK, CHUNK)
    rel_t = _gather_t(relation_table, idx2d)
    sub_t, obj_t = _passthrough_t(subject_embeddings.T, object_embeddings.T)
    return (sub_t.T, rel_t.T, obj_t.T)


# speedup vs baseline: 1.3614x; 1.3614x over previous
import functools

import jax
import jax.numpy as jnp
from jax import lax
from jax.experimental import pallas as pl
from jax.experimental.pallas import tpu as pltpu
from jax.experimental.pallas import tpu_sc as plsc

BATCH = 16384
DIM = 64
LANES = 16
NUM_CORES = 2
NUM_SUBCORES = 16
NUM_WORKERS = NUM_CORES * NUM_SUBCORES
ROWS_PER_WORKER = BATCH // NUM_WORKERS
CHUNK = 128
NCHUNK = ROWS_PER_WORKER // CHUNK
NGROUP = ROWS_PER_WORKER // LANES

COPY_GRID = 8
COPY_BLOCK = BATCH // COPY_GRID


def _gather_body(table_hbm, idx_hbm, out_hbm, idx_v, rows_v, out_t_v, sem):
    wid = lax.axis_index("s") * NUM_CORES + lax.axis_index("c")
    base = wid * ROWS_PER_WORKER
    pltpu.sync_copy(idx_hbm.at[pl.ds(wid * NCHUNK, NCHUNK)], idx_v)
    gathers = [
        pltpu.async_copy(
            table_hbm.at[idx_v.at[j]],
            rows_v.at[pl.ds(j * CHUNK, CHUNK)],
            sem,
        )
        for j in range(NCHUNK)
    ]
    for g in gathers:
        g.wait()

    dvecs = [
        dg * LANES + lax.iota(jnp.int32, LANES) for dg in range(DIM // LANES)
    ]

    @plsc.parallel_loop(0, ROWS_PER_WORKER)
    def _(b):
        bvec = jnp.full((LANES,), b, dtype=jnp.int32)
        for dg in range(DIM // LANES):
            vals = rows_v[b, pl.ds(dg * LANES, LANES)]
            plsc.store_scatter(out_t_v, [dvecs[dg], bvec], vals)

    pltpu.sync_copy(
        out_t_v.at[:, pl.ds(0, ROWS_PER_WORKER)],
        out_hbm.at[:, pl.ds(base, ROWS_PER_WORKER)],
    )


_gather_t = functools.partial(
    pl.kernel,
    out_type=jax.ShapeDtypeStruct((DIM, BATCH), jnp.float32),
    mesh=plsc.VectorSubcoreMesh(core_axis_name="c", subcore_axis_name="s"),
    scratch_types=[
        pltpu.VMEM((NCHUNK, CHUNK), jnp.int32),
        pltpu.VMEM((ROWS_PER_WORKER, DIM), jnp.float32),
        pltpu.VMEM((DIM, ROWS_PER_WORKER + 1), jnp.float32),
        pltpu.SemaphoreType.DMA,
    ],
    compiler_params=pltpu.CompilerParams(
        use_tc_tiling_on_sc=False, needs_layout_passes=False
    ),
)(_gather_body)


def _copy_body(sub_ref, obj_ref, sub_out_ref, obj_out_ref):
    sub_out_ref[...] = sub_ref[...]
    obj_out_ref[...] = obj_ref[...]


_passthrough_t = pl.pallas_call(
    _copy_body,
    grid=(COPY_GRID,),
    in_specs=[
        pl.BlockSpec((DIM, COPY_BLOCK), lambda i: (0, i)),
        pl.BlockSpec((DIM, COPY_BLOCK), lambda i: (0, i)),
    ],
    out_specs=[
        pl.BlockSpec((DIM, COPY_BLOCK), lambda i: (0, i)),
        pl.BlockSpec((DIM, COPY_BLOCK), lambda i: (0, i)),
    ],
    out_shape=(
        jax.ShapeDtypeStruct((DIM, BATCH), jnp.float32),
        jax.ShapeDtypeStruct((DIM, BATCH), jnp.float32),
    ),
)


def kernel(subject_embeddings, relation_ids, object_embeddings, relation_table):
    idx2d = relation_ids.astype(jnp.int32).reshape(NUM_WORKERS * NCHUNK, CHUNK)
    rel_t = _gather_t(relation_table, idx2d)
    sub_t, obj_t = _passthrough_t(subject_embeddings.T, object_embeddings.T)
    return (sub_t.T, rel_t.T, obj_t.T)
